# baseline (speedup 1.0000x reference)
import functools

import jax
import jax.numpy as jnp
from jax import lax
from jax.experimental import pallas as pl
from jax.experimental.pallas import tpu as pltpu
from jax.experimental.pallas import tpu_sc as plsc

N_NODES = 10000
N_EDGES = 320000
D = 128

NC = 2
NS = 16
NW = NC * NS

EDGES_PER_TILE = N_EDGES // NW
CHUNK = 40
N_CHUNKS = EDGES_PER_TILE // CHUNK

ROW_CHUNK = 80
N_ROW_CHUNKS = N_NODES // ROW_CHUNK
ROW_CHUNKS_PER_TILE = -(-N_ROW_CHUNKS // NS)

ROW_BLK = 1000
N_BLK = N_NODES // ROW_BLK


def _mm_body(x_ref, w1_ref, w2_ref, y_ref):
    h = jax.lax.dot_general(x_ref[...], w1_ref[...], (((1,), (1,)), ((), ())),
                            precision=lax.Precision.HIGHEST,
                            preferred_element_type=jnp.float32)
    y_ref[...] = jax.lax.dot_general(h, w2_ref[...], (((1,), (1,)), ((), ())),
                                     precision=lax.Precision.HIGHEST,
                                     preferred_element_type=jnp.float32)


def _combine_body(p_ref, bias_ref, w2_ref, b2_ref, out_ref):
    c = jax.lax.dot_general(bias_ref[...], w2_ref[...], (((1,), (1,)), ((), ())),
                            precision=lax.Precision.HIGHEST,
                            preferred_element_type=jnp.float32) + b2_ref[...]
    out_ref[...] = p_ref[0] + p_ref[1] + c


def _sc_body(src_hbm, dst_hbm, y_hbm, out_hbm, acc, idx_s, idx_d, rows, buf,
             sem_g):
    cc = lax.axis_index("c")
    ss = lax.axis_index("s")
    wid = cc * NS + ss

    def zero_row(i, _):
        for j in range(D // 16):
            buf[i, pl.ds(j * 16, 16)] = jnp.zeros((16,), jnp.float32)
        return _
    lax.fori_loop(0, ROW_CHUNK, zero_row, None)
    for k in range(ROW_CHUNKS_PER_TILE):
        cid = ss * ROW_CHUNKS_PER_TILE + k

        @pl.when(cid < N_ROW_CHUNKS)
        def _():
            r0 = pl.multiple_of(cid * ROW_CHUNK, ROW_CHUNK)
            pltpu.sync_copy(buf, acc.at[pl.ds(r0, ROW_CHUNK), :])
    plsc.subcore_barrier()

    base_e = wid * EDGES_PER_TILE

    def edge_chunk(t, _):
        e0 = pl.multiple_of(base_e + t * CHUNK, CHUNK)
        pltpu.sync_copy(src_hbm.at[pl.ds(e0, CHUNK)], idx_s)
        pltpu.sync_copy(dst_hbm.at[pl.ds(e0, CHUNK)], idx_d)
        pltpu.async_copy(y_hbm.at[idx_s], rows, sem_g).wait()
        pltpu.sync_copy(rows, acc.at[idx_d], add=True)
        return _
    lax.fori_loop(0, N_CHUNKS, edge_chunk, None)
    plsc.subcore_barrier()

    for k in range(ROW_CHUNKS_PER_TILE):
        cid = ss * ROW_CHUNKS_PER_TILE + k

        @pl.when(cid < N_ROW_CHUNKS)
        def _():
            r0 = pl.multiple_of(cid * ROW_CHUNK, ROW_CHUNK)
            pltpu.sync_copy(acc.at[pl.ds(r0, ROW_CHUNK), :], buf)
            pltpu.sync_copy(buf, out_hbm.at[cc, pl.ds(r0, ROW_CHUNK), :])


def _sc_propagate(src, dst, y):
    mesh = plsc.VectorSubcoreMesh(core_axis_name="c", subcore_axis_name="s",
                                  num_cores=NC, num_subcores=NS)
    f = pl.kernel(
        _sc_body,
        out_type=jax.ShapeDtypeStruct((NC, N_NODES, D), jnp.float32),
        mesh=mesh,
        scratch_types=[
            pltpu.VMEM_SHARED((N_NODES, D), jnp.float32),
            pltpu.VMEM((CHUNK,), jnp.int32),
            pltpu.VMEM((CHUNK,), jnp.int32),
            pltpu.VMEM((CHUNK, D), jnp.float32),
            pltpu.VMEM((ROW_CHUNK, D), jnp.float32),
            pltpu.SemaphoreType.DMA,
        ],
    )
    return f(src, dst, y)


def kernel(x, edge_index, W1, bias, W2, b2):
    src = edge_index[0]
    dst = edge_index[1]

    y = pl.pallas_call(
        _mm_body,
        grid=(N_BLK,),
        in_specs=[
            pl.BlockSpec((ROW_BLK, D), lambda i: (i, 0)),
            pl.BlockSpec((D, D), lambda i: (0, 0)),
            pl.BlockSpec((D, D), lambda i: (0, 0)),
        ],
        out_specs=pl.BlockSpec((ROW_BLK, D), lambda i: (i, 0)),
        out_shape=jax.ShapeDtypeStruct((N_NODES, D), jnp.float32),
    )(x, W1, W2)

    p = _sc_propagate(src, dst, y)

    out = pl.pallas_call(
        _combine_body,
        grid=(N_BLK,),
        in_specs=[
            pl.BlockSpec((NC, ROW_BLK, D), lambda i: (0, i, 0)),
            pl.BlockSpec((1, D), lambda i: (0, 0)),
            pl.BlockSpec((D, D), lambda i: (0, 0)),
            pl.BlockSpec((1, D), lambda i: (0, 0)),
        ],
        out_specs=pl.BlockSpec((ROW_BLK, D), lambda i: (i, 0)),
        out_shape=jax.ShapeDtypeStruct((N_NODES, D), jnp.float32),
    )(p, bias[None, :], W2, b2[None, :])

    return out


# speedup vs baseline: 2.0275x; 2.0275x over previous
import functools

import jax
import jax.numpy as jnp
from jax import lax
from jax.experimental import pallas as pl
from jax.experimental.pallas import tpu as pltpu
from jax.experimental.pallas import tpu_sc as plsc

N_NODES = 10000
N_EDGES = 320000
D = 128

NC = 2
NS = 16
NW = NC * NS

EDGES_PER_TILE = N_EDGES // NW
CHUNK = 80
N_CHUNKS = EDGES_PER_TILE // CHUNK

ROW_CHUNK = 80
N_ROW_CHUNKS = N_NODES // ROW_CHUNK
ROW_CHUNKS_PER_TILE = -(-N_ROW_CHUNKS // NS)

ROW_BLK = 1000
N_BLK = N_NODES // ROW_BLK


def _mm_body(x_ref, w1_ref, w2_ref, y_ref):
    h = jax.lax.dot_general(x_ref[...], w1_ref[...], (((1,), (1,)), ((), ())),
                            precision=lax.Precision.HIGHEST,
                            preferred_element_type=jnp.float32)
    y_ref[...] = jax.lax.dot_general(h, w2_ref[...], (((1,), (1,)), ((), ())),
                                     precision=lax.Precision.HIGHEST,
                                     preferred_element_type=jnp.float32)


def _combine_body(p_ref, bias_ref, w2_ref, b2_ref, out_ref):
    c = jax.lax.dot_general(bias_ref[...], w2_ref[...], (((1,), (1,)), ((), ())),
                            precision=lax.Precision.HIGHEST,
                            preferred_element_type=jnp.float32) + b2_ref[...]
    out_ref[...] = p_ref[0] + p_ref[1] + c


def _sc_body(src_hbm, dst_hbm, y_hbm, out_hbm, acc, idx_s, idx_d, rows, buf,
             sem_g, sem_i):
    cc = lax.axis_index("c")
    ss = lax.axis_index("s")
    wid = cc * NS + ss

    def zero_row(i, _):
        for j in range(D // 16):
            buf[i, pl.ds(j * 16, 16)] = jnp.zeros((16,), jnp.float32)
        return _
    lax.fori_loop(0, ROW_CHUNK, zero_row, None)
    for k in range(ROW_CHUNKS_PER_TILE):
        cid = ss * ROW_CHUNKS_PER_TILE + k

        @pl.when(cid < N_ROW_CHUNKS)
        def _():
            r0 = pl.multiple_of(cid * ROW_CHUNK, ROW_CHUNK)
            pltpu.sync_copy(buf, acc.at[pl.ds(r0, ROW_CHUNK), :])
    plsc.subcore_barrier()

    base_e = wid * EDGES_PER_TILE
    pltpu.sync_copy(src_hbm.at[pl.ds(pl.multiple_of(base_e, CHUNK), CHUNK)],
                    idx_s.at[0])
    pltpu.sync_copy(dst_hbm.at[pl.ds(pl.multiple_of(base_e, CHUNK), CHUNK)],
                    idx_d.at[0])

    def edge_chunk(t, _):
        p = lax.rem(t, 2)
        pn = 1 - p
        g = pltpu.async_copy(y_hbm.at[idx_s.at[p]], rows, sem_g)

        @pl.when(t < N_CHUNKS - 1)
        def _():
            e1 = pl.multiple_of(base_e + (t + 1) * CHUNK, CHUNK)
            i1 = pltpu.async_copy(src_hbm.at[pl.ds(e1, CHUNK)],
                                  idx_s.at[pn], sem_i)
            i2 = pltpu.async_copy(dst_hbm.at[pl.ds(e1, CHUNK)],
                                  idx_d.at[pn], sem_i)
        g.wait()
        pltpu.sync_copy(rows, acc.at[idx_d.at[p]], add=True)

        @pl.when(t < N_CHUNKS - 1)
        def _():
            e1 = pl.multiple_of(base_e + (t + 1) * CHUNK, CHUNK)
            pltpu.make_async_copy(src_hbm.at[pl.ds(e1, CHUNK)],
                                  idx_s.at[pn], sem_i).wait()
            pltpu.make_async_copy(dst_hbm.at[pl.ds(e1, CHUNK)],
                                  idx_d.at[pn], sem_i).wait()
        return _
    lax.fori_loop(0, N_CHUNKS, edge_chunk, None)
    plsc.subcore_barrier()

    for k in range(ROW_CHUNKS_PER_TILE):
        cid = ss * ROW_CHUNKS_PER_TILE + k

        @pl.when(cid < N_ROW_CHUNKS)
        def _():
            r0 = pl.multiple_of(cid * ROW_CHUNK, ROW_CHUNK)
            pltpu.sync_copy(acc.at[pl.ds(r0, ROW_CHUNK), :], buf)
            pltpu.sync_copy(buf, out_hbm.at[cc, pl.ds(r0, ROW_CHUNK), :])


def _sc_propagate(src, dst, y):
    mesh = plsc.VectorSubcoreMesh(core_axis_name="c", subcore_axis_name="s",
                                  num_cores=NC, num_subcores=NS)
    f = pl.kernel(
        _sc_body,
        out_type=jax.ShapeDtypeStruct((NC, N_NODES, D), jnp.float32),
        mesh=mesh,
        scratch_types=[
            pltpu.VMEM_SHARED((N_NODES, D), jnp.float32),
            pltpu.VMEM((2, CHUNK), jnp.int32),
            pltpu.VMEM((2, CHUNK), jnp.int32),
            pltpu.VMEM((CHUNK, D), jnp.float32),
            pltpu.VMEM((ROW_CHUNK, D), jnp.float32),
            pltpu.SemaphoreType.DMA,
            pltpu.SemaphoreType.DMA,
        ],
    )
    return f(src, dst, y)


def kernel(x, edge_index, W1, bias, W2, b2):
    src = edge_index[0]
    dst = edge_index[1]

    y = pl.pallas_call(
        _mm_body,
        grid=(N_BLK,),
        in_specs=[
            pl.BlockSpec((ROW_BLK, D), lambda i: (i, 0)),
            pl.BlockSpec((D, D), lambda i: (0, 0)),
            pl.BlockSpec((D, D), lambda i: (0, 0)),
        ],
        out_specs=pl.BlockSpec((ROW_BLK, D), lambda i: (i, 0)),
        out_shape=jax.ShapeDtypeStruct((N_NODES, D), jnp.float32),
    )(x, W1, W2)

    p = _sc_propagate(src, dst, y)

    out = pl.pallas_call(
        _combine_body,
        grid=(N_BLK,),
        in_specs=[
            pl.BlockSpec((NC, ROW_BLK, D), lambda i: (0, i, 0)),
            pl.BlockSpec((1, D), lambda i: (0, 0)),
            pl.BlockSpec((D, D), lambda i: (0, 0)),
            pl.BlockSpec((1, D), lambda i: (0, 0)),
        ],
        out_specs=pl.BlockSpec((ROW_BLK, D), lambda i: (i, 0)),
        out_shape=jax.ShapeDtypeStruct((N_NODES, D), jnp.float32),
    )(p, bias[None, :], W2, b2[None, :])

    return out


# speedup vs baseline: 2.0319x; 1.0022x over previous
import functools

import jax
import jax.numpy as jnp
from jax import lax
from jax.experimental import pallas as pl
from jax.experimental.pallas import tpu as pltpu
from jax.experimental.pallas import tpu_sc as plsc

N_NODES = 10000
N_EDGES = 320000
D = 128

NC = 2
NS = 16
NW = NC * NS

EDGES_PER_TILE = N_EDGES // NW
CHUNK = 80
N_CHUNKS = EDGES_PER_TILE // CHUNK

ROW_CHUNK = 80
N_ROW_CHUNKS = N_NODES // ROW_CHUNK
ROW_CHUNKS_PER_TILE = -(-N_ROW_CHUNKS // NS)

ROW_BLK = 1000
N_BLK = N_NODES // ROW_BLK


def _mm_body(x_ref, w1_ref, w2_ref, y_ref):
    h = jax.lax.dot_general(x_ref[...], w1_ref[...], (((1,), (1,)), ((), ())),
                            precision=lax.Precision.HIGHEST,
                            preferred_element_type=jnp.float32)
    y_ref[...] = jax.lax.dot_general(h, w2_ref[...], (((1,), (1,)), ((), ())),
                                     precision=lax.Precision.HIGHEST,
                                     preferred_element_type=jnp.float32)


def _combine_body(p_ref, bias_ref, w2_ref, b2_ref, out_ref):
    c = jax.lax.dot_general(bias_ref[...], w2_ref[...], (((1,), (1,)), ((), ())),
                            precision=lax.Precision.HIGHEST,
                            preferred_element_type=jnp.float32) + b2_ref[...]
    out_ref[...] = p_ref[0] + p_ref[1] + c


def _sc_body(src_hbm, dst_hbm, y_hbm, out_hbm, acc, idx_s, idx_d, rows, buf,
             sem_g, sem_i, sem_s):
    cc = lax.axis_index("c")
    ss = lax.axis_index("s")
    wid = cc * NS + ss

    def zero_row(i, _):
        for j in range(D // 16):
            buf[i, pl.ds(j * 16, 16)] = jnp.zeros((16,), jnp.float32)
        return _
    lax.fori_loop(0, ROW_CHUNK, zero_row, None)
    for k in range(ROW_CHUNKS_PER_TILE):
        cid = ss * ROW_CHUNKS_PER_TILE + k

        @pl.when(cid < N_ROW_CHUNKS)
        def _():
            r0 = pl.multiple_of(cid * ROW_CHUNK, ROW_CHUNK)
            pltpu.sync_copy(buf, acc.at[pl.ds(r0, ROW_CHUNK), :])
    plsc.subcore_barrier()

    base_e = wid * EDGES_PER_TILE
    pltpu.sync_copy(src_hbm.at[pl.ds(pl.multiple_of(base_e, CHUNK), CHUNK)],
                    idx_s.at[0])
    pltpu.sync_copy(dst_hbm.at[pl.ds(pl.multiple_of(base_e, CHUNK), CHUNK)],
                    idx_d.at[0])

    def edge_chunk(t, _):
        p = lax.rem(t, 2)
        pn = 1 - p

        @pl.when(t >= 1)
        def _():
            pltpu.make_async_copy(y_hbm.at[pl.ds(0, CHUNK)], rows.at[pn],
                                  sem_s).wait()
        g = pltpu.async_copy(y_hbm.at[idx_s.at[p]], rows.at[p], sem_g)

        @pl.when(t < N_CHUNKS - 1)
        def _():
            e1 = pl.multiple_of(base_e + (t + 1) * CHUNK, CHUNK)
            pltpu.async_copy(src_hbm.at[pl.ds(e1, CHUNK)],
                             idx_s.at[pn], sem_i)
            pltpu.async_copy(dst_hbm.at[pl.ds(e1, CHUNK)],
                             idx_d.at[pn], sem_i)
        g.wait()
        pltpu.async_copy(rows.at[p], acc.at[idx_d.at[p]], sem_s, add=True)

        @pl.when(t < N_CHUNKS - 1)
        def _():
            e1 = pl.multiple_of(base_e + (t + 1) * CHUNK, CHUNK)
            pltpu.make_async_copy(src_hbm.at[pl.ds(e1, CHUNK)],
                                  idx_s.at[pn], sem_i).wait()
            pltpu.make_async_copy(dst_hbm.at[pl.ds(e1, CHUNK)],
                                  idx_d.at[pn], sem_i).wait()
        return _
    lax.fori_loop(0, N_CHUNKS, edge_chunk, None)
    pltpu.make_async_copy(y_hbm.at[pl.ds(0, CHUNK)], rows.at[0],
                          sem_s).wait()
    plsc.subcore_barrier()

    for k in range(ROW_CHUNKS_PER_TILE):
        cid = ss * ROW_CHUNKS_PER_TILE + k

        @pl.when(cid < N_ROW_CHUNKS)
        def _():
            r0 = pl.multiple_of(cid * ROW_CHUNK, ROW_CHUNK)
            pltpu.sync_copy(acc.at[pl.ds(r0, ROW_CHUNK), :], buf)
            pltpu.sync_copy(buf, out_hbm.at[cc, pl.ds(r0, ROW_CHUNK), :])


def _sc_propagate(src, dst, y):
    mesh = plsc.VectorSubcoreMesh(core_axis_name="c", subcore_axis_name="s",
                                  num_cores=NC, num_subcores=NS)
    f = pl.kernel(
        _sc_body,
        out_type=jax.ShapeDtypeStruct((NC, N_NODES, D), jnp.float32),
        mesh=mesh,
        scratch_types=[
            pltpu.VMEM_SHARED((N_NODES, D), jnp.float32),
            pltpu.VMEM((2, CHUNK), jnp.int32),
            pltpu.VMEM((2, CHUNK), jnp.int32),
            pltpu.VMEM((2, CHUNK, D), jnp.float32),
            pltpu.VMEM((ROW_CHUNK, D), jnp.float32),
            pltpu.SemaphoreType.DMA,
            pltpu.SemaphoreType.DMA,
            pltpu.SemaphoreType.DMA,
        ],
    )
    return f(src, dst, y)


def kernel(x, edge_index, W1, bias, W2, b2):
    src = edge_index[0]
    dst = edge_index[1]

    y = pl.pallas_call(
        _mm_body,
        grid=(N_BLK,),
        in_specs=[
            pl.BlockSpec((ROW_BLK, D), lambda i: (i, 0)),
            pl.BlockSpec((D, D), lambda i: (0, 0)),
            pl.BlockSpec((D, D), lambda i: (0, 0)),
        ],
        out_specs=pl.BlockSpec((ROW_BLK, D), lambda i: (i, 0)),
        out_shape=jax.ShapeDtypeStruct((N_NODES, D), jnp.float32),
    )(x, W1, W2)

    p = _sc_propagate(src, dst, y)

    out = pl.pallas_call(
        _combine_body,
        grid=(N_BLK,),
        in_specs=[
            pl.BlockSpec((NC, ROW_BLK, D), lambda i: (0, i, 0)),
            pl.BlockSpec((1, D), lambda i: (0, 0)),
            pl.BlockSpec((D, D), lambda i: (0, 0)),
            pl.BlockSpec((1, D), lambda i: (0, 0)),
        ],
        out_specs=pl.BlockSpec((ROW_BLK, D), lambda i: (i, 0)),
        out_shape=jax.ShapeDtypeStruct((N_NODES, D), jnp.float32),
    )(p, bias[None, :], W2, b2[None, :])

    return out
